# Initial kernel scaffold; baseline (speedup 1.0000x reference)
import jax
import jax.numpy as jnp
from jax.experimental import pallas as pl


def kernel(x, edge_index, batch, Wl1, bl1, Wr1, br1, p1, Wl2, bl2, Wr2, br2, p2, Wl3, bl3, Wr3, br3, p3, W1, b1, W2, b2, W3, b3):
    raise NotImplementedError("write your pallas kernel here")


# speedup vs baseline: 9.4651x; 9.4651x over previous
import functools

import jax
import jax.numpy as jnp
import numpy as np
from jax import lax
from jax.experimental import pallas as pl
from jax.experimental.pallas import tpu as pltpu
from jax.experimental.pallas import tpu_sc as plsc

N = 10000
E = 320000
F = 128
NPAD = 10240
NROW = 80
NC = 2
NS = 16
NW = NC * NS
WIN = 128
EPT = 10112
NWIN = EPT // WIN
EPAD = EPT * NW
SLC = NPAD // NS
INT_MIN = np.int32(-2147483648)


def _make_segsum(masked: bool):
    mesh = plsc.VectorSubcoreMesh(core_axis_name="c", subcore_axis_name="s")
    out_type = [
        jax.ShapeDtypeStruct((NC, NPAD, F), jnp.float32),
        jax.ShapeDtypeStruct((NC, NPAD), jnp.float32),
    ]
    scratch = [
        pltpu.VMEM((WIN,), jnp.int32),
        pltpu.VMEM((WIN,), jnp.int32),
        pltpu.VMEM((WIN,), jnp.int32),
        pltpu.VMEM((WIN,), jnp.int32),
        pltpu.VMEM((WIN,), jnp.int32),
        pltpu.VMEM((WIN, F), jnp.float32),
        pltpu.VMEM((WIN,), jnp.float32),
        pltpu.VMEM((WIN,), jnp.float32),
        pltpu.VMEM((16, F), jnp.float32),
        pltpu.VMEM_SHARED((NPAD, F), jnp.float32),
        pltpu.VMEM_SHARED((NPAD,), jnp.float32),
        pltpu.VMEM_SHARED((NPAD,), jnp.int32),
        pltpu.SemaphoreType.DMA,
        pltpu.SemaphoreType.DMA,
    ]

    def body(z_hbm, src_hbm, dst_hbm, kept_hbm, agg_out, cnt_out,
             src_v, dst_v, ks_v, kd_v, de_v, rows_v, ones_v, zrow_v, zblk_v,
             acc_sp, cnt_sp, kept_sp, sem0, sem1):
        cid = lax.axis_index("c")
        sid = lax.axis_index("s")
        wid = cid * NS + sid

        zeros16 = jnp.zeros((16,), jnp.float32)
        for g in range(WIN // 16):
            ones_v[pl.ds(g * 16, 16)] = jnp.ones((16,), jnp.float32)
            zrow_v[pl.ds(g * 16, 16)] = zeros16
        for r in range(16):
            for g in range(F // 16):
                zblk_v[r, pl.ds(g * 16, 16)] = zeros16

        base_r = sid * SLC

        @pl.loop(0, SLC // 16)
        def _(i):
            pltpu.sync_copy(zblk_v, acc_sp.at[pl.ds(base_r + i * 16, 16)])

        for i in range(SLC // WIN):
            pltpu.sync_copy(zrow_v, cnt_sp.at[pl.ds(base_r + i * WIN, WIN)])
        if masked:
            pltpu.sync_copy(kept_hbm.at[pl.ds(base_r, SLC)],
                            kept_sp.at[pl.ds(base_r, SLC)])
        plsc.subcore_barrier()

        ebase = wid * EPT

        @pl.loop(0, NWIN)
        def _(j):
            off = ebase + j * WIN
            pltpu.sync_copy(src_hbm.at[pl.ds(off, WIN)], src_v)
            pltpu.sync_copy(dst_hbm.at[pl.ds(off, WIN)], dst_v)
            gat = pltpu.async_copy(z_hbm.at[src_v], rows_v, sem0)
            if masked:
                pltpu.async_copy(kept_sp.at[src_v], ks_v, sem1).wait()
                pltpu.async_copy(kept_sp.at[dst_v], kd_v, sem1).wait()
                for i in range(WIN // 16):
                    ks = ks_v[pl.ds(i * 16, 16)]
                    kd = kd_v[pl.ds(i * 16, 16)]
                    d = dst_v[pl.ds(i * 16, 16)]
                    valid = (ks > 0) & (kd > 0)
                    dummy = jnp.int32(N + i * 16) + lax.iota(jnp.int32, 16)
                    de_v[pl.ds(i * 16, 16)] = jnp.where(valid, d, dummy)
                idx = de_v
            else:
                idx = dst_v
            gat.wait()
            pltpu.sync_copy(rows_v, acc_sp.at[idx], add=True)
            pltpu.sync_copy(ones_v, cnt_sp.at[idx], add=True)

        plsc.subcore_barrier()

        pltpu.sync_copy(acc_sp.at[pl.ds(base_r, SLC)],
                        agg_out.at[cid, pl.ds(base_r, SLC)])
        pltpu.sync_copy(cnt_sp.at[pl.ds(base_r, SLC)],
                        cnt_out.at[cid, pl.ds(base_r, SLC)])

    return pl.kernel(body, out_type=out_type, mesh=mesh, scratch_types=scratch)


_segsum_masked = _make_segsum(True)
_segsum_nomask = _make_segsum(False)


def _monokey(s):
    b = lax.bitcast_convert_type(s, jnp.int32)
    return jnp.where(b < 0, jnp.bitwise_xor(jnp.bitwise_not(b), INT_MIN), b)


def _count(mask):
    return jnp.sum(mask.astype(jnp.int32))


def _select_topk(skey, k):
    def vbody(i, t):
        cand = t + (jnp.int32(1) << (31 - i))
        return jnp.where(_count(skey >= cand) >= k, cand, t)
    t = lax.fori_loop(0, 32, vbody, INT_MIN)
    eq = skey == t
    m = k - _count(skey > t)
    gidx = (lax.broadcasted_iota(jnp.int32, (NROW, F), 0) * F
            + lax.broadcasted_iota(jnp.int32, (NROW, F), 1))

    def ibody(i, v):
        cand = v + (jnp.int32(1) << (13 - i))
        return jnp.where(_count(eq & (gidx < cand)) < m, cand, v)
    v = lax.fori_loop(0, 14, ibody, jnp.int32(0))
    return t, v


def _select_topk(skey, k):
    def vbody(i, t):
        cand = t + (jnp.int32(1) << (31 - i))
        return jnp.where(_count(skey >= cand) >= k, cand, t)
    t = lax.fori_loop(0, 32, vbody, INT_MIN)
    eq = skey == t
    m = k - _count(skey > t)
    gidx = (lax.broadcasted_iota(jnp.int32, (NROW, F), 0) * F
            + lax.broadcasted_iota(jnp.int32, (NROW, F), 1))

    def ibody(i, v):
        cand = v + (jnp.int32(1) << (13 - i))
        return jnp.where(_count(eq & (gidx < cand)) < m, cand, v)
    v = lax.fori_loop(0, 14, ibody, jnp.int32(0))
    return t, v


BR = 1280
GR = NPAD // BR


def _tc_matmul_body(agg_ref, cnt_ref, z_ref, actc_ref, Wl_ref, bl_ref, Wr_ref,
                    br_ref, p_ref, h_ref, s_ref, skey_ref):
    c = cnt_ref[0] + cnt_ref[1]
    mean = (agg_ref[0] + agg_ref[1]) / jnp.maximum(c, 1.0)
    h = jnp.dot(mean, Wl_ref[...], preferred_element_type=jnp.float32)
    h = h + jnp.dot(z_ref[...], Wr_ref[...], preferred_element_type=jnp.float32)
    h = jnp.maximum(h + bl_ref[...] + br_ref[...], 0.0)
    p = p_ref[...]
    pn = p * lax.rsqrt(jnp.sum(p * p))
    s = jnp.tanh(jnp.dot(h, jnp.reshape(pn, (F, 1)),
                         preferred_element_type=jnp.float32))
    h_ref[...] = h
    s_ref[...] = s
    skey_ref[...] = jnp.where(actc_ref[...] > 0, _monokey(s), INT_MIN)


_tc_matmul = pl.pallas_call(
    _tc_matmul_body,
    grid=(GR,),
    in_specs=[
        pl.BlockSpec((NC, BR, F), lambda i: (0, i, 0)),
        pl.BlockSpec((NC, BR, 1), lambda i: (0, i, 0)),
        pl.BlockSpec((BR, F), lambda i: (i, 0)),
        pl.BlockSpec((BR, 1), lambda i: (i, 0)),
        pl.BlockSpec((F, F), lambda i: (0, 0)),
        pl.BlockSpec((F,), lambda i: (0,)),
        pl.BlockSpec((F, F), lambda i: (0, 0)),
        pl.BlockSpec((F,), lambda i: (0,)),
        pl.BlockSpec((F,), lambda i: (0,)),
    ],
    out_specs=[
        pl.BlockSpec((BR, F), lambda i: (i, 0)),
        pl.BlockSpec((BR, 1), lambda i: (i, 0)),
        pl.BlockSpec((BR, 1), lambda i: (i, 0)),
    ],
    out_shape=[
        jax.ShapeDtypeStruct((NPAD, F), jnp.float32),
        jax.ShapeDtypeStruct((NPAD, 1), jnp.float32),
        jax.ShapeDtypeStruct((NPAD, 1), jnp.int32),
    ],
)


def _make_tc_pool(k):
    def body(h_ref, s_ref, skey_ref, zout_ref, kept_ref, ro_ref):
        skey_c = skey_ref[...]
        skey80 = jnp.reshape(skey_c, (NROW, F))
        t, v = _select_topk(skey80, k)
        gidx80 = (lax.broadcasted_iota(jnp.int32, (NROW, F), 0) * F
                  + lax.broadcasted_iota(jnp.int32, (NROW, F), 1))
        kept80 = (skey80 > t) | ((skey80 == t) & (gidx80 <= v))
        gidx_c = lax.broadcasted_iota(jnp.int32, (NPAD, 1), 0)
        keptc = (skey_c > t) | ((skey_c == t) & (gidx_c <= v))
        z_new = jnp.where(keptc, h_ref[...] * s_ref[...], 0.0)
        gmp = jnp.max(jnp.where(keptc, z_new, -jnp.inf), axis=0, keepdims=True)
        gap = jnp.sum(z_new, axis=0, keepdims=True) * (1.0 / k)
        zout_ref[...] = z_new
        kept_ref[...] = kept80.astype(jnp.int32)
        ro_ref[...] = jnp.concatenate([gmp, gap], axis=1)

    return pl.pallas_call(
        body,
        out_shape=[
            jax.ShapeDtypeStruct((NPAD, F), jnp.float32),
            jax.ShapeDtypeStruct((NROW, F), jnp.int32),
            jax.ShapeDtypeStruct((1, 2 * F), jnp.float32),
        ],
    )


def _tc_mlp_body(r1_ref, r2_ref, r3_ref, W1_ref, b1_ref, W2_ref, b2_ref,
                 W3_ref, b3_ref, out_ref):
    h0 = r1_ref[...] + r2_ref[...] + r3_ref[...]
    t1 = jnp.maximum(jnp.dot(h0, W1_ref[...],
                             preferred_element_type=jnp.float32) + b1_ref[...],
                     0.0)
    t2 = jnp.maximum(jnp.dot(t1, W2_ref[...],
                             preferred_element_type=jnp.float32) + b2_ref[...],
                     0.0)
    lg = jnp.dot(t2, W3_ref[...],
                 preferred_element_type=jnp.float32) + b3_ref[...]
    e = lg - jnp.max(lg, axis=1, keepdims=True)
    out_ref[...] = e - jnp.log(jnp.sum(jnp.exp(e), axis=1, keepdims=True))


_tc_mlp = pl.pallas_call(
    _tc_mlp_body,
    out_shape=jax.ShapeDtypeStruct((1, 10), jnp.float32),
)

_tc_pool1 = _make_tc_pool(5000)
_tc_pool2 = _make_tc_pool(2500)
_tc_pool3 = _make_tc_pool(1250)


def kernel(x, edge_index, batch, Wl1, bl1, Wr1, br1, p1, Wl2, bl2, Wr2, br2,
           p2, Wl3, bl3, Wr3, br3, p3, W1, b1, W2, b2, W3, b3):
    z = jnp.pad(x, ((0, NPAD - N), (0, 0)))
    pad_n = EPAD - E
    src_p = jnp.concatenate(
        [edge_index[0], jnp.zeros((pad_n,), jnp.int32)])
    dst_p = jnp.concatenate(
        [edge_index[1], N + (jnp.arange(pad_n, dtype=jnp.int32) % (NPAD - N))])
    act = (jnp.arange(NPAD, dtype=jnp.int32) < N).astype(jnp.int32)

    ros = []
    actc = act.reshape(NPAD, 1)
    kept_flat = act
    for li, (Wl, bl, Wr, br, p, pool) in enumerate([
            (Wl1, bl1, Wr1, br1, p1, _tc_pool1),
            (Wl2, bl2, Wr2, br2, p2, _tc_pool2),
            (Wl3, bl3, Wr3, br3, p3, _tc_pool3)]):
        seg = _segsum_nomask if li == 0 else _segsum_masked
        agg, cnt = seg(z, src_p, dst_p, kept_flat)
        h, s, skey = _tc_matmul(agg, cnt.reshape(NC, NPAD, 1), z, actc,
                                Wl, bl, Wr, br, p)
        z, kept80, ro = pool(h, s, skey)
        ros.append(ro)
        actc = kept80.reshape(NPAD, 1)
        kept_flat = kept80.reshape(NPAD)
    return _tc_mlp(ros[0], ros[1], ros[2], W1, b1, W2, b2, W3, b3)
